# baseline (speedup 1.0000x reference)
import functools

import jax
import jax.numpy as jnp
from jax import lax
from jax.experimental import pallas as pl
from jax.experimental.pallas import tpu as pltpu
from jax.experimental.pallas import tpu_sc as plsc

_VOCAB = 100000
_EMB = 128
_B = 1024
_L = 200
_N = _B * _L

_NUM_CORES = 2
_NUM_SUBCORES = 16
_NW = _NUM_CORES * _NUM_SUBCORES
_PER_W = _N // _NW
_CHUNK = 128
_NCH = _PER_W // _CHUNK
_LANES = 16
_DEPTH = 5
_AHEAD = 3


def _make_kernel():
    mesh = plsc.VectorSubcoreMesh(core_axis_name="c", subcore_axis_name="s")

    scratch = [
        pltpu.VMEM((_PER_W,), jnp.int32),
        pltpu.VMEM((_PER_W,), jnp.int32),
        pltpu.VMEM((3, _EMB), jnp.float32),
    ]
    scratch += [pltpu.VMEM((_CHUNK, _EMB), jnp.float32) for _ in range(_DEPTH)]
    scratch += [pltpu.SemaphoreType.DMA for _ in range(2 * _DEPTH)]

    @functools.partial(
        pl.kernel,
        mesh=mesh,
        out_type=jax.ShapeDtypeStruct((_N, _EMB), jnp.float32),
        scratch_types=scratch,
    )
    def emb_kernel(table_hbm, seg_hbm, x_hbm, lbl_hbm, out_hbm,
                   idx_v, lbl_v, seg_local, *rest):
        bufs = rest[:_DEPTH]
        semg = rest[_DEPTH:2 * _DEPTH]
        semo = rest[2 * _DEPTH:3 * _DEPTH]

        wid = lax.axis_index("s") * _NUM_CORES + lax.axis_index("c")
        ob = wid * _PER_W

        pltpu.sync_copy(x_hbm.at[pl.ds(ob, _PER_W)], idx_v)
        pltpu.sync_copy(lbl_hbm.at[pl.ds(ob, _PER_W)], lbl_v)
        pltpu.sync_copy(seg_hbm, seg_local)

        def gather_start(g, j):
            pltpu.make_async_copy(
                table_hbm.at[idx_v.at[pl.ds(g * _CHUNK, _CHUNK)]], bufs[j], semg[j]).start()

        def gather_wait(g, j):
            pltpu.make_async_copy(
                table_hbm.at[idx_v.at[pl.ds(g * _CHUNK, _CHUNK)]], bufs[j], semg[j]).wait()

        def out_start(g, j):
            pltpu.make_async_copy(
                bufs[j], out_hbm.at[pl.ds(ob + g * _CHUNK, _CHUNK)],
                semo[j]).start()

        def out_wait(j):
            pltpu.make_async_copy(
                bufs[j], out_hbm.at[pl.ds(ob, _CHUNK)], semo[j]).wait()

        def add_seg(g, j):
            buf = bufs[j]
            seg_rows = [
                [seg_local[r, pl.ds(c * _LANES, _LANES)]
                 for c in range(_EMB // _LANES)]
                for r in range(3)
            ]

            @pl.loop(0, _CHUNK, step=_LANES)
            def _(i0):
                lab = lbl_v[pl.ds(g * _CHUNK + i0, _LANES)]
                for k in range(_LANES):
                    l = lab[k]
                    is1 = l == 1
                    is2 = l == 2
                    for c in range(_EMB // _LANES):
                        s = pl.ds(c * _LANES, _LANES)
                        seg_c = jnp.where(
                            is2, seg_rows[2][c],
                            jnp.where(is1, seg_rows[1][c], seg_rows[0][c]))
                        buf[i0 + k, s] = buf[i0 + k, s] + seg_c

        for g0 in range(_AHEAD):
            gather_start(g0, g0)

        def body(g, j):
            jn = (j + _AHEAD) % _DEPTH

            @pl.when(g >= _DEPTH - _AHEAD)
            def _():
                out_wait(jn)

            @pl.when(g + _AHEAD < _NCH)
            def _():
                gather_start(g + _AHEAD, jn)

            gather_wait(g, j)
            add_seg(g, j)
            out_start(g, j)

        @pl.loop(0, _NCH, step=_DEPTH)
        def _(h):
            for jj in range(_DEPTH):
                body(h + jj, jj)

        for g0 in range(_NCH - (_DEPTH - _AHEAD), _NCH):
            out_wait(g0 % _DEPTH)

    return emb_kernel


_emb_kernel = _make_kernel()


@jax.jit
def kernel(x, segmet_label, table, seg_table):
    x2 = x.reshape(_N).astype(jnp.int32)
    lbl2 = segmet_label.reshape(_N).astype(jnp.int32)
    out = _emb_kernel(table, seg_table, x2, lbl2)
    return out.reshape(_B, _L, _EMB)


# speedup vs baseline: 1.0003x; 1.0003x over previous
import functools

import jax
import jax.numpy as jnp
from jax import lax
from jax.experimental import pallas as pl
from jax.experimental.pallas import tpu as pltpu
from jax.experimental.pallas import tpu_sc as plsc

_VOCAB = 100000
_EMB = 128
_B = 1024
_L = 200
_N = _B * _L

_NUM_CORES = 2
_NUM_SUBCORES = 16
_NW = _NUM_CORES * _NUM_SUBCORES
_PER_W = _N // _NW
_CHUNK = 128
_NCH = _PER_W // _CHUNK
_LANES = 16
_DEPTH = 5
_AHEAD = 3
_NSTREAM = 2
_SUB = _CHUNK // _NSTREAM


def _make_kernel():
    mesh = plsc.VectorSubcoreMesh(core_axis_name="c", subcore_axis_name="s")

    scratch = [
        pltpu.VMEM((_PER_W,), jnp.int32),
        pltpu.VMEM((_PER_W,), jnp.int32),
        pltpu.VMEM((3, _EMB), jnp.float32),
    ]
    scratch += [pltpu.VMEM((_CHUNK, _EMB), jnp.float32) for _ in range(_DEPTH)]
    scratch += [pltpu.SemaphoreType.DMA for _ in range(2 * _DEPTH)]

    @functools.partial(
        pl.kernel,
        mesh=mesh,
        out_type=jax.ShapeDtypeStruct((_N, _EMB), jnp.float32),
        scratch_types=scratch,
    )
    def emb_kernel(table_hbm, seg_hbm, x_hbm, lbl_hbm, out_hbm,
                   idx_v, lbl_v, seg_local, *rest):
        bufs = rest[:_DEPTH]
        semg = rest[_DEPTH:2 * _DEPTH]
        semo = rest[2 * _DEPTH:3 * _DEPTH]

        wid = lax.axis_index("s") * _NUM_CORES + lax.axis_index("c")
        ob = wid * _PER_W

        pltpu.sync_copy(x_hbm.at[pl.ds(ob, _PER_W)], idx_v)
        pltpu.sync_copy(lbl_hbm.at[pl.ds(ob, _PER_W)], lbl_v)
        pltpu.sync_copy(seg_hbm, seg_local)

        def gather_start(g, j):
            for h in range(_NSTREAM):
                pltpu.make_async_copy(
                    table_hbm.at[idx_v.at[pl.ds(g * _CHUNK + h * _SUB, _SUB)]],
                    bufs[j].at[pl.ds(h * _SUB, _SUB)], semg[j]).start()

        def gather_wait(g, j):
            for h in range(_NSTREAM):
                pltpu.make_async_copy(
                    table_hbm.at[idx_v.at[pl.ds(g * _CHUNK + h * _SUB, _SUB)]],
                    bufs[j].at[pl.ds(h * _SUB, _SUB)], semg[j]).wait()

        def out_start(g, j):
            pltpu.make_async_copy(
                bufs[j], out_hbm.at[pl.ds(ob + g * _CHUNK, _CHUNK)],
                semo[j]).start()

        def out_wait(j):
            pltpu.make_async_copy(
                bufs[j], out_hbm.at[pl.ds(ob, _CHUNK)], semo[j]).wait()

        def add_seg(g, j):
            buf = bufs[j]
            seg_rows = [
                [seg_local[r, pl.ds(c * _LANES, _LANES)]
                 for c in range(_EMB // _LANES)]
                for r in range(3)
            ]

            @pl.loop(0, _CHUNK, step=_LANES)
            def _(i0):
                lab = lbl_v[pl.ds(g * _CHUNK + i0, _LANES)]
                for k in range(_LANES):
                    l = lab[k]
                    is1 = l == 1
                    is2 = l == 2
                    for c in range(_EMB // _LANES):
                        s = pl.ds(c * _LANES, _LANES)
                        seg_c = jnp.where(
                            is2, seg_rows[2][c],
                            jnp.where(is1, seg_rows[1][c], seg_rows[0][c]))
                        buf[i0 + k, s] = buf[i0 + k, s] + seg_c

        for g0 in range(_AHEAD):
            gather_start(g0, g0)

        def body(g, j):
            jn = (j + _AHEAD) % _DEPTH

            @pl.when(g >= _DEPTH - _AHEAD)
            def _():
                out_wait(jn)

            @pl.when(g + _AHEAD < _NCH)
            def _():
                gather_start(g + _AHEAD, jn)

            gather_wait(g, j)
            add_seg(g, j)
            out_start(g, j)

        @pl.loop(0, _NCH, step=_DEPTH)
        def _(h):
            for jj in range(_DEPTH):
                body(h + jj, jj)

        for g0 in range(_NCH - (_DEPTH - _AHEAD), _NCH):
            out_wait(g0 % _DEPTH)

    return emb_kernel


_emb_kernel = _make_kernel()


@jax.jit
def kernel(x, segmet_label, table, seg_table):
    x2 = x.reshape(_N).astype(jnp.int32)
    lbl2 = segmet_label.reshape(_N).astype(jnp.int32)
    out = _emb_kernel(table, seg_table, x2, lbl2)
    return out.reshape(_B, _L, _EMB)


# speedup vs baseline: 1.0094x; 1.0091x over previous
import functools

import jax
import jax.numpy as jnp
from jax import lax
from jax.experimental import pallas as pl
from jax.experimental.pallas import tpu as pltpu
from jax.experimental.pallas import tpu_sc as plsc

_VOCAB = 100000
_EMB = 128
_B = 1024
_L = 200
_N = _B * _L

_NUM_CORES = 2
_NUM_SUBCORES = 16
_NW = _NUM_CORES * _NUM_SUBCORES
_PER_W = _N // _NW
_CHUNK = 128
_NCH = _PER_W // _CHUNK
_LANES = 16
_DEPTH = 5
_AHEAD = 3
_NSTREAM = 2
_SUB = _CHUNK // _NSTREAM


def _make_kernel():
    mesh = plsc.VectorSubcoreMesh(core_axis_name="c", subcore_axis_name="s")

    scratch = [
        pltpu.VMEM((_PER_W,), jnp.int32),
        pltpu.VMEM((_PER_W,), jnp.int32),
        pltpu.VMEM((3, _EMB), jnp.float32),
    ]
    scratch += [pltpu.VMEM((_CHUNK, _EMB), jnp.float32) for _ in range(_DEPTH)]
    scratch += [pltpu.SemaphoreType.DMA for _ in range(2 * _DEPTH + 1)]

    @functools.partial(
        pl.kernel,
        mesh=mesh,
        out_type=jax.ShapeDtypeStruct((_N, _EMB), jnp.float32),
        scratch_types=scratch,
    )
    def emb_kernel(table_hbm, seg_hbm, x_hbm, lbl_hbm, out_hbm,
                   idx_v, lbl_v, seg_local, *rest):
        bufs = rest[:_DEPTH]
        semg = rest[_DEPTH:2 * _DEPTH]
        semo = rest[2 * _DEPTH:3 * _DEPTH]
        semp = rest[3 * _DEPTH]

        wid = lax.axis_index("s") * _NUM_CORES + lax.axis_index("c")
        ob = wid * _PER_W

        cp_idx = pltpu.make_async_copy(x_hbm.at[pl.ds(ob, _PER_W)], idx_v, semp)
        cp_lbl = pltpu.make_async_copy(lbl_hbm.at[pl.ds(ob, _PER_W)], lbl_v, semp)
        cp_seg = pltpu.make_async_copy(seg_hbm, seg_local, semp)
        cp_idx.start()
        cp_lbl.start()
        cp_seg.start()
        cp_idx.wait()
        cp_lbl.wait()
        cp_seg.wait()

        def gather_start(g, j):
            for h in range(_NSTREAM):
                pltpu.make_async_copy(
                    table_hbm.at[idx_v.at[pl.ds(g * _CHUNK + h * _SUB, _SUB)]],
                    bufs[j].at[pl.ds(h * _SUB, _SUB)], semg[j]).start()

        def gather_wait(g, j):
            for h in range(_NSTREAM):
                pltpu.make_async_copy(
                    table_hbm.at[idx_v.at[pl.ds(g * _CHUNK + h * _SUB, _SUB)]],
                    bufs[j].at[pl.ds(h * _SUB, _SUB)], semg[j]).wait()

        def out_start(g, j):
            pltpu.make_async_copy(
                bufs[j], out_hbm.at[pl.ds(ob + g * _CHUNK, _CHUNK)],
                semo[j]).start()

        def out_wait(j):
            pltpu.make_async_copy(
                bufs[j], out_hbm.at[pl.ds(ob, _CHUNK)], semo[j]).wait()

        def add_seg(g, j):
            buf = bufs[j]
            seg_rows = [
                [seg_local[r, pl.ds(c * _LANES, _LANES)]
                 for c in range(_EMB // _LANES)]
                for r in range(3)
            ]

            @pl.loop(0, _CHUNK, step=_LANES)
            def _(i0):
                lab = lbl_v[pl.ds(g * _CHUNK + i0, _LANES)]
                for k in range(_LANES):
                    l = lab[k]
                    is1 = l == 1
                    is2 = l == 2
                    for c in range(_EMB // _LANES):
                        s = pl.ds(c * _LANES, _LANES)
                        seg_c = jnp.where(
                            is2, seg_rows[2][c],
                            jnp.where(is1, seg_rows[1][c], seg_rows[0][c]))
                        buf[i0 + k, s] = buf[i0 + k, s] + seg_c

        for g0 in range(_AHEAD):
            gather_start(g0, g0)

        def body(g, j):
            jn = (j + _AHEAD) % _DEPTH

            @pl.when(g >= _DEPTH - _AHEAD)
            def _():
                out_wait(jn)

            @pl.when(g + _AHEAD < _NCH)
            def _():
                gather_start(g + _AHEAD, jn)

            gather_wait(g, j)
            add_seg(g, j)
            out_start(g, j)

        @pl.loop(0, _NCH, step=_DEPTH)
        def _(h):
            for jj in range(_DEPTH):
                body(h + jj, jj)

        for g0 in range(_NCH - (_DEPTH - _AHEAD), _NCH):
            out_wait(g0 % _DEPTH)

    return emb_kernel


_emb_kernel = _make_kernel()


@jax.jit
def kernel(x, segmet_label, table, seg_table):
    x2 = x.reshape(_N).astype(jnp.int32)
    lbl2 = segmet_label.reshape(_N).astype(jnp.int32)
    out = _emb_kernel(table, seg_table, x2, lbl2)
    return out.reshape(_B, _L, _EMB)
